# baseline (speedup 1.0000x reference)
import jax
import jax.numpy as jnp
from jax.experimental import pallas as pl
from jax.experimental.pallas import tpu as pltpu

_COLS = 1024
_BR = 512


def _tanh_block(x_ref, o_ref):
    o_ref[...] = jnp.tanh(x_ref[...])


def kernel(mean, stddev):
    del stddev
    m, n = mean.shape
    rows = (m * n) // _COLS
    x = mean.reshape(rows, _COLS)
    out = pl.pallas_call(
        _tanh_block,
        grid=(pl.cdiv(rows, _BR),),
        in_specs=[pl.BlockSpec((_BR, _COLS), lambda i: (i, 0))],
        out_specs=pl.BlockSpec((_BR, _COLS), lambda i: (i, 0)),
        out_shape=jax.ShapeDtypeStruct((rows, _COLS), jnp.float32),
    )(x)
    return out.reshape(m, n)


# speedup vs baseline: 2.9948x; 2.9948x over previous
import jax
import jax.numpy as jnp
from jax.experimental import pallas as pl
from jax.experimental.pallas import tpu as pltpu

_BK = 4096


def _tanh_block(x_ref, o_ref):
    o_ref[...] = jnp.tanh(x_ref[...])


def kernel(mean, stddev):
    del stddev
    m, n = mean.shape
    return pl.pallas_call(
        _tanh_block,
        grid=(pl.cdiv(n, _BK),),
        in_specs=[pl.BlockSpec((m, _BK), lambda j: (0, j))],
        out_specs=pl.BlockSpec((m, _BK), lambda j: (0, j)),
        out_shape=jax.ShapeDtypeStruct((m, n), jnp.float32),
    )(mean)


# speedup vs baseline: 3.0907x; 1.0320x over previous
import jax
import jax.numpy as jnp
from jax.experimental import pallas as pl
from jax.experimental.pallas import tpu as pltpu

_NCHUNK = 16
_RING = 8


def _body(x_hbm, o_hbm, *scratch):
    bufs = scratch[:_RING]
    isems = scratch[_RING]
    osems = scratch[_RING + 1]
    rb = x_hbm.shape[0] // _NCHUNK

    def in_copy(c, s):
        return pltpu.make_async_copy(
            x_hbm.at[pl.ds(c * rb, rb), :], bufs[s], isems.at[s])

    def out_copy(c, s):
        return pltpu.make_async_copy(
            bufs[s], o_hbm.at[pl.ds(c * rb, rb), :], osems.at[s])

    for c in range(_RING):
        in_copy(c, c).start()
    for c in range(_NCHUNK):
        s = c % _RING
        in_copy(c, s).wait()
        bufs[s][...] = jnp.tanh(bufs[s][...])
        out_copy(c, s).start()
        nc = c + _RING
        if nc < _NCHUNK:
            out_copy(c, s).wait()
            in_copy(nc, s).start()
    for c in range(_NCHUNK - _RING, _NCHUNK):
        out_copy(c, c % _RING).wait()


def kernel(mean, stddev):
    del stddev
    m, n = mean.shape
    rb = m // _NCHUNK
    return pl.pallas_call(
        _body,
        in_specs=[pl.BlockSpec(memory_space=pl.ANY)],
        out_specs=pl.BlockSpec(memory_space=pl.ANY),
        out_shape=jax.ShapeDtypeStruct((m, n), jnp.float32),
        scratch_shapes=(
            [pltpu.VMEM((rb, n), jnp.float32) for _ in range(_RING)]
            + [pltpu.SemaphoreType.DMA((_RING,)),
               pltpu.SemaphoreType.DMA((_RING,))]
        ),
    )(mean)
